# baseline (speedup 1.0000x reference)
import functools

import jax
import jax.numpy as jnp
from jax import lax
from jax.experimental import pallas as pl
from jax.experimental.pallas import tpu as pltpu
from jax.experimental.pallas import tpu_sc as plsc

N = 10000
F_IN = 128
H = 256
HH = 128
G = 64

BLK = 1024
NP = 10240
NB = 10

E_REAL = 330000
NEG_INF = float('-inf')


def _l1_body(x_ref, w_ref, as_ref, ad_ref, h_ref, av_ref, ms_ref):
    i = pl.program_id(0)
    c = pl.program_id(1)
    x = x_ref[...]
    h = jnp.dot(x, w_ref[...], preferred_element_type=jnp.float32)
    h_ref[...] = h
    pa_s = jnp.sum(h * as_ref[...], axis=1, keepdims=True)
    pa_d = jnp.sum(h * ad_ref[...], axis=1, keepdims=True)
    part = jnp.concatenate([pa_s, pa_d], axis=1)

    @pl.when(c == 0)
    def _():
        av_ref[...] = part

    @pl.when(c == 1)
    def _():
        av = av_ref[...] + part
        av_ref[...] = av
        row = i * BLK + lax.broadcasted_iota(jnp.int32, (BLK, 2), 0)
        valid = row < N
        masked = jnp.where(valid, av, NEG_INF)
        cur = jnp.max(masked, axis=0, keepdims=True)
        prev = jnp.where(i == 0, jnp.full((1, 2), NEG_INF), ms_ref[...])
        ms_ref[...] = jnp.maximum(prev, cur)


def _l2_body(lo_ref, hi_ref, w_ref, as_ref, ad_ref, sc_ref, sh_ref,
             h_ref, av_ref, ms_ref):
    i = pl.program_id(0)
    c = pl.program_id(1)
    flo = jnp.maximum(lo_ref[...] * sc_ref[:, :HH] + sh_ref[:, :HH], 0.0)
    fhi = jnp.maximum(hi_ref[...] * sc_ref[:, HH:] + sh_ref[:, HH:], 0.0)
    w = w_ref[...]
    h = (jnp.dot(flo, w[:HH, :], preferred_element_type=jnp.float32)
         + jnp.dot(fhi, w[HH:, :], preferred_element_type=jnp.float32))
    h_ref[...] = h
    pa_s = jnp.sum(h * as_ref[...], axis=1, keepdims=True)
    pa_d = jnp.sum(h * ad_ref[...], axis=1, keepdims=True)
    part = jnp.concatenate([pa_s, pa_d], axis=1)

    @pl.when(c == 0)
    def _():
        av_ref[...] = part

    @pl.when(c == 1)
    def _():
        av = av_ref[...] + part
        av_ref[...] = av
        row = i * BLK + lax.broadcasted_iota(jnp.int32, (BLK, 2), 0)
        valid = row < N
        masked = jnp.where(valid, av, NEG_INF)
        cur = jnp.max(masked, axis=0, keepdims=True)
        prev = jnp.where(i == 0, jnp.full((1, 2), NEG_INF), ms_ref[...])
        ms_ref[...] = jnp.maximum(prev, cur)


def _run_l1(x_pad, W1, att_s, att_d):
    return pl.pallas_call(
        _l1_body,
        grid=(NB, 2),
        in_specs=[
            pl.BlockSpec((BLK, F_IN), lambda i, c: (i, 0)),
            pl.BlockSpec((F_IN, HH), lambda i, c: (0, c)),
            pl.BlockSpec((1, HH), lambda i, c: (0, c)),
            pl.BlockSpec((1, HH), lambda i, c: (0, c)),
        ],
        out_specs=[
            pl.BlockSpec((BLK, HH), lambda i, c: (c * NB + i, 0)),
            pl.BlockSpec((BLK, 2), lambda i, c: (i, 0)),
            pl.BlockSpec((1, 2), lambda i, c: (0, 0)),
        ],
        out_shape=[
            jax.ShapeDtypeStruct((2 * NP, HH), jnp.float32),
            jax.ShapeDtypeStruct((NP, 2), jnp.float32),
            jax.ShapeDtypeStruct((1, 2), jnp.float32),
        ],
    )(x_pad, W1, att_s, att_d)


def _run_l2(h_flat, W2, att_s, att_d, scale, shift):
    return pl.pallas_call(
        _l2_body,
        grid=(NB, 2),
        in_specs=[
            pl.BlockSpec((BLK, HH), lambda i, c: (i, 0)),
            pl.BlockSpec((BLK, HH), lambda i, c: (NB + i, 0)),
            pl.BlockSpec((H, HH), lambda i, c: (0, c)),
            pl.BlockSpec((1, HH), lambda i, c: (0, c)),
            pl.BlockSpec((1, HH), lambda i, c: (0, c)),
            pl.BlockSpec((1, H), lambda i, c: (0, 0)),
            pl.BlockSpec((1, H), lambda i, c: (0, 0)),
        ],
        out_specs=[
            pl.BlockSpec((BLK, HH), lambda i, c: (c * NB + i, 0)),
            pl.BlockSpec((BLK, 2), lambda i, c: (i, 0)),
            pl.BlockSpec((1, 2), lambda i, c: (0, 0)),
        ],
        out_shape=[
            jax.ShapeDtypeStruct((2 * NP, HH), jnp.float32),
            jax.ShapeDtypeStruct((NP, 2), jnp.float32),
            jax.ShapeDtypeStruct((1, 2), jnp.float32),
        ],
    )(h_flat, h_flat, W2, att_s, att_d, scale, shift)


def _stats_body(lo_ref, hi_ref, s_ref, ss_ref):
    i = pl.program_id(0)

    @pl.when(i == 0)
    def _():
        s_ref[...] = jnp.zeros_like(s_ref)
        ss_ref[...] = jnp.zeros_like(ss_ref)

    blk = jnp.concatenate([lo_ref[...], hi_ref[...]], axis=1)
    s_ref[...] += jnp.sum(blk, axis=0, keepdims=True)
    ss_ref[...] += jnp.sum(blk * blk, axis=0, keepdims=True)


def _run_stats(flat):
    return pl.pallas_call(
        _stats_body,
        grid=(NB,),
        in_specs=[
            pl.BlockSpec((BLK, HH), lambda i: (i, 0)),
            pl.BlockSpec((BLK, HH), lambda i: (NB + i, 0)),
        ],
        out_specs=[
            pl.BlockSpec((1, H), lambda i: (0, 0)),
            pl.BlockSpec((1, H), lambda i: (0, 0)),
        ],
        out_shape=[
            jax.ShapeDtypeStruct((1, H), jnp.float32),
            jax.ShapeDtypeStruct((1, H), jnp.float32),
        ],
    )(flat, flat)


def _final_body(lo_ref, hi_ref, sc_ref, sh_ref, b_ref, l1w_ref, l1b_ref,
                l2w_ref, l2b_ref, out_ref, plo_ref, phi_ref, cnt_ref):
    i = pl.program_id(0)

    @pl.when(i == 0)
    def _():
        plo_ref[...] = jnp.zeros_like(plo_ref)
        phi_ref[...] = jnp.zeros_like(phi_ref)
        cnt_ref[...] = jnp.zeros_like(cnt_ref)

    flo = jnp.maximum(lo_ref[...] * sc_ref[:, :HH] + sh_ref[:, :HH], 0.0)
    fhi = jnp.maximum(hi_ref[...] * sc_ref[:, HH:] + sh_ref[:, HH:], 0.0)
    bv = b_ref[...].reshape(1, BLK)
    gi = lax.broadcasted_iota(jnp.int32, (G, BLK), 0).astype(jnp.float32)
    oh = (gi == bv).astype(jnp.float32)
    plo_ref[...] += jnp.dot(oh, flo, preferred_element_type=jnp.float32)
    phi_ref[...] += jnp.dot(oh, fhi, preferred_element_type=jnp.float32)
    cnt_ref[...] += jnp.broadcast_to(jnp.sum(oh, axis=1, keepdims=True), (G, HH))

    @pl.when(i == NB - 1)
    def _():
        cc = jnp.maximum(cnt_ref[...], 1.0)
        p_lo = plo_ref[...] / cc
        p_hi = phi_ref[...] / cc
        l1w = l1w_ref[...]
        hm = jnp.maximum(
            jnp.dot(p_lo, l1w[:HH, :], preferred_element_type=jnp.float32)
            + jnp.dot(p_hi, l1w[HH:, :], preferred_element_type=jnp.float32)
            + l1b_ref[...], 0.0)
        out_ref[...] = (jnp.dot(hm, l2w_ref[...],
                                preferred_element_type=jnp.float32)
                        + l2b_ref[...])


def _run_final(flat, scale, shift, batch3, lin1_W, lin1_b, lin2_W, lin2_b):
    return pl.pallas_call(
        _final_body,
        grid=(NB,),
        in_specs=[
            pl.BlockSpec((BLK, HH), lambda i: (i, 0)),
            pl.BlockSpec((BLK, HH), lambda i: (NB + i, 0)),
            pl.BlockSpec((1, H), lambda i: (0, 0)),
            pl.BlockSpec((1, H), lambda i: (0, 0)),
            pl.BlockSpec((1, 1, BLK), lambda i: (i, 0, 0)),
            pl.BlockSpec((H, HH), lambda i: (0, 0)),
            pl.BlockSpec((1, HH), lambda i: (0, 0)),
            pl.BlockSpec((HH, 1), lambda i: (0, 0)),
            pl.BlockSpec((1, 1), lambda i: (0, 0)),
        ],
        out_specs=pl.BlockSpec((G, 1), lambda i: (0, 0)),
        out_shape=jax.ShapeDtypeStruct((G, 1), jnp.float32),
        scratch_shapes=[
            pltpu.VMEM((G, HH), jnp.float32),
            pltpu.VMEM((G, HH), jnp.float32),
            pltpu.VMEM((G, HH), jnp.float32),
        ],
    )(flat, flat, scale, shift, batch3, lin1_W, lin1_b, lin2_W, lin2_b)


SE = 48
SR = 32
SCK = 14
ET = SCK * SR * SE
EPAD = 16 * ET
EROWS = EPAD // SE
FC = 40


def _sc_body(src_hbm, dst_hbm, asrc_hbm, adst_hbm, cvec_hbm, h_hbm, out_hbm,
             as_v, ad_v, src_s, dst_s, w_s, den_c, cvec_v,
             rows0, rows1, out_sh, den_sh, gsem0, gsem1, ssem0, ssem1, dsem):
    c = lax.axis_index("c")
    t = lax.axis_index("s")
    zrows = NP // 16

    pltpu.sync_copy(asrc_hbm, as_v)
    pltpu.sync_copy(adst_hbm, ad_v)
    pltpu.sync_copy(cvec_hbm, cvec_v)
    cv = cvec_v[...]

    z16 = jnp.zeros((16,), jnp.float32)

    def _zero_row(r, carry):
        for q in range(HH // 16):
            rows0[r, pl.ds(q * 16, 16)] = z16
        return carry

    lax.fori_loop(0, FC, _zero_row, 0, unroll=4)
    for k in range(SE // 16):
        den_c[pl.ds(k * 16, 16)] = z16

    def _zchunk(q, carry):
        base = t * zrows + q * FC
        pltpu.sync_copy(rows0.at[pl.ds(0, FC)], out_sh.at[pl.ds(base, FC)])
        pltpu.sync_copy(den_c.at[pl.ds(0, FC)], den_sh.at[pl.ds(base, FC)])
        return carry

    lax.fori_loop(0, zrows // FC, _zchunk, 0)
    plsc.subcore_barrier()

    def _stage_and_ex(s, add_off):
        row0 = t * (SCK * SR) + s * SR
        pltpu.sync_copy(src_hbm.at[pl.ds(row0, SR)], src_s)
        pltpu.sync_copy(dst_hbm.at[pl.ds(row0, SR)], dst_s)
        ebase = t * ET + s * (SR * SE)

        def _row(r, carry):
            for k in range(SE // 16):
                s16 = src_s[r, pl.ds(k * 16, 16)]
                d16 = dst_s[r, pl.ds(k * 16, 16)]
                a1 = plsc.load_gather(as_v, [s16])
                a2 = plsc.load_gather(ad_v, [d16])
                al = a1 + a2
                al = jnp.where(al > 0.0, al, al * jnp.float32(0.2))
                ex = jnp.exp(al - cv)
                eid = (ebase + r * SE + k * 16
                       + lax.broadcasted_iota(jnp.int32, (16,), 0))
                ex = jnp.where(eid < E_REAL, ex, jnp.float32(0.0))
                w_s[r, pl.ds(k * 16, 16)] = ex
                if add_off:
                    src_s[r, pl.ds(k * 16, 16)] = s16 + c * NP
            return carry

        lax.fori_loop(0, SR, _row, 0, unroll=4)

    def _mul_batch(r, rows):
        def _edge(j, carry):
            cb = plsc.load_gather(
                w_s, [jnp.full((16,), r, jnp.int32),
                      jnp.full((16,), j, jnp.int32)])
            for q in range(HH // 16):
                rows[j, pl.ds(q * 16, 16)] = rows[j, pl.ds(q * 16, 16)] * cb
            return carry

        lax.fori_loop(0, SE, _edge, 0, unroll=8)

    def _p2(s, carry):
        _stage_and_ex(s, True)
        pltpu.async_copy(h_hbm.at[src_s.at[0]], rows0, gsem0)
        pltpu.async_copy(h_hbm.at[src_s.at[1]], rows1, gsem1)

        def _fire(r, carry2):
            pltpu.async_copy(w_s.at[r], den_sh.at[dst_s.at[r]], dsem,
                             add=True)
            return carry2

        lax.fori_loop(0, SR, _fire, 0, unroll=8)

        def _pair(rr, carry2):
            r = rr * 2
            pltpu.make_async_copy(h_hbm.at[src_s.at[r]], rows0, gsem0).wait()
            _mul_batch(r, rows0)
            pltpu.async_copy(rows0, out_sh.at[dst_s.at[r]], ssem0, add=True)

            pltpu.make_async_copy(h_hbm.at[src_s.at[r + 1]], rows1,
                                  gsem1).wait()
            _mul_batch(r + 1, rows1)
            pltpu.async_copy(rows1, out_sh.at[dst_s.at[r + 1]], ssem1,
                             add=True)

            pltpu.make_async_copy(rows0, out_sh.at[dst_s.at[r]],
                                  ssem0).wait()

            @pl.when(r + 2 < SR)
            def _():
                pltpu.async_copy(h_hbm.at[src_s.at[r + 2]], rows0, gsem0)

            pltpu.make_async_copy(rows1, out_sh.at[dst_s.at[r + 1]],
                                  ssem1).wait()

            @pl.when(r + 3 < SR)
            def _():
                pltpu.async_copy(h_hbm.at[src_s.at[r + 3]], rows1, gsem1)

            return carry2

        lax.fori_loop(0, SR // 2, _pair, 0)

        def _drain(r, carry2):
            pltpu.make_async_copy(w_s.at[r], den_sh.at[dst_s.at[r]],
                                  dsem).wait()
            return carry2

        lax.fori_loop(0, SR, _drain, 0, unroll=4)
        return carry

    lax.fori_loop(0, SCK, _p2, 0)
    plsc.subcore_barrier()

    def _fin_chunk(q, carry):
        base = t * zrows + q * FC
        pltpu.sync_copy(out_sh.at[pl.ds(base, FC)], rows0.at[pl.ds(0, FC)])
        pltpu.sync_copy(den_sh.at[pl.ds(base, FC)], den_c.at[pl.ds(0, FC)])

        def _rowdiv(r, carry2):
            dv = plsc.load_gather(den_c, [jnp.full((16,), r, jnp.int32)])
            dv = dv + jnp.float32(1e-16)
            for qq in range(HH // 16):
                rows0[r, pl.ds(qq * 16, 16)] = rows0[r, pl.ds(qq * 16, 16)] / dv
            return carry2

        lax.fori_loop(0, FC, _rowdiv, 0, unroll=4)
        pltpu.sync_copy(rows0.at[pl.ds(0, FC)],
                        out_hbm.at[pl.ds(c * NP + base, FC)])
        return carry

    lax.fori_loop(0, zrows // FC, _fin_chunk, 0)


@functools.lru_cache(maxsize=1)
def _sc_agg():
    return functools.partial(
        pl.kernel,
        out_type=jax.ShapeDtypeStruct((2 * NP, HH), jnp.float32),
        mesh=plsc.VectorSubcoreMesh(core_axis_name="c", subcore_axis_name="s"),
        compiler_params=pltpu.CompilerParams(needs_layout_passes=False),
        scratch_types=[
            pltpu.VMEM((NP,), jnp.float32),
            pltpu.VMEM((NP,), jnp.float32),
            pltpu.VMEM((SR, SE), jnp.int32),
            pltpu.VMEM((SR, SE), jnp.int32),
            pltpu.VMEM((SR, SE), jnp.float32),
            pltpu.VMEM((SE,), jnp.float32),
            pltpu.VMEM((16,), jnp.float32),
            pltpu.VMEM((SE, HH), jnp.float32),
            pltpu.VMEM((SE, HH), jnp.float32),
            pltpu.VMEM_SHARED((NP, HH), jnp.float32),
            pltpu.VMEM_SHARED((NP,), jnp.float32),
            pltpu.SemaphoreType.DMA,
            pltpu.SemaphoreType.DMA,
            pltpu.SemaphoreType.DMA,
            pltpu.SemaphoreType.DMA,
            pltpu.SemaphoreType.DMA,
        ],
    )(_sc_body)


def _sc_layer(src2d, dst2d, asrc, adst, cvec, h_flat):
    return _sc_agg()(src2d, dst2d, asrc, adst, cvec, h_flat)


def _bn_params(s, ss, g, b):
    mu = s[0] / N
    var = ss[0] / N - mu * mu
    scale = g / jnp.sqrt(var + 1e-5)
    shift = b - mu * scale
    return scale.reshape(1, H), shift.reshape(1, H)


def kernel(x, edge_index, batch, W1, att_src1, att_dst1, b1, bn1_g, bn1_b,
           W2, att_src2, att_dst2, b2, bn2_g, bn2_b,
           lin1_W, lin1_b, lin2_W, lin2_b):
    loop = jnp.arange(N, dtype=edge_index.dtype)
    src = jnp.concatenate([edge_index[0], loop])
    dst = jnp.concatenate([edge_index[1], loop])
    src2d = jnp.pad(src, (0, EPAD - E_REAL)).reshape(EROWS, SE)
    dst2d = jnp.pad(dst, (0, EPAD - E_REAL)).reshape(EROWS, SE)

    x_pad = jnp.pad(x, ((0, NP - N), (0, 0)))
    batch3 = jnp.pad(batch.astype(jnp.float32), (0, NP - N),
                     constant_values=-1.0).reshape(NB, 1, BLK)

    h1_flat, av1, ms1 = _run_l1(x_pad, W1, att_src1.reshape(1, H),
                                att_dst1.reshape(1, H))
    c1 = jnp.full((16,), ms1[0, 0] + ms1[0, 1], jnp.float32)
    asrc1 = jnp.pad(av1[:N, 0], (0, NP - N))
    adst1 = jnp.pad(av1[:N, 1], (0, NP - N))
    agg1 = _sc_layer(src2d, dst2d, asrc1, adst1, c1, h1_flat)

    s1, ss1 = _run_stats(agg1)
    scale1, shift1 = _bn_params(s1, ss1, bn1_g, bn1_b)

    h2_flat, av2, ms2 = _run_l2(agg1, W2, att_src2.reshape(1, H),
                                att_dst2.reshape(1, H), scale1, shift1)
    c2 = jnp.full((16,), ms2[0, 0] + ms2[0, 1], jnp.float32)
    asrc2 = jnp.pad(av2[:N, 0], (0, NP - N))
    adst2 = jnp.pad(av2[:N, 1], (0, NP - N))
    agg2 = _sc_layer(src2d, dst2d, asrc2, adst2, c2, h2_flat)

    s2, ss2 = _run_stats(agg2)
    scale2, shift2 = _bn_params(s2, ss2, bn2_g, bn2_b)

    out = _run_final(agg2, scale2, shift2, batch3,
                     lin1_W, lin1_b.reshape(1, HH),
                     lin2_W, lin2_b.reshape(1, 1))
    return out.reshape(-1)


# speedup vs baseline: 1.0895x; 1.0895x over previous
import functools

import jax
import jax.numpy as jnp
from jax import lax
from jax.experimental import pallas as pl
from jax.experimental.pallas import tpu as pltpu
from jax.experimental.pallas import tpu_sc as plsc

N = 10000
F_IN = 128
H = 256
HH = 128
G = 64

BLK = 1024
NP = 10240
NB = 10

E_REAL = 330000
NEG_INF = float('-inf')


def _l1_body(x_ref, w_ref, as_ref, ad_ref, h_ref, av_ref, ms_ref):
    i = pl.program_id(0)
    c = pl.program_id(1)
    x = x_ref[...]
    h = jnp.dot(x, w_ref[...], preferred_element_type=jnp.float32)
    h_ref[...] = h
    pa_s = jnp.sum(h * as_ref[...], axis=1, keepdims=True)
    pa_d = jnp.sum(h * ad_ref[...], axis=1, keepdims=True)
    part = jnp.concatenate([pa_s, pa_d], axis=1)

    @pl.when(c == 0)
    def _():
        av_ref[...] = part

    @pl.when(c == 1)
    def _():
        av = av_ref[...] + part
        av_ref[...] = av
        row = i * BLK + lax.broadcasted_iota(jnp.int32, (BLK, 2), 0)
        valid = row < N
        masked = jnp.where(valid, av, NEG_INF)
        cur = jnp.max(masked, axis=0, keepdims=True)
        prev = jnp.where(i == 0, jnp.full((1, 2), NEG_INF), ms_ref[...])
        ms_ref[...] = jnp.maximum(prev, cur)


def _l2_body(lo_ref, hi_ref, w_ref, as_ref, ad_ref, sc_ref, sh_ref,
             h_ref, av_ref, ms_ref):
    i = pl.program_id(0)
    c = pl.program_id(1)
    flo = jnp.maximum(lo_ref[...] * sc_ref[:, :HH] + sh_ref[:, :HH], 0.0)
    fhi = jnp.maximum(hi_ref[...] * sc_ref[:, HH:] + sh_ref[:, HH:], 0.0)
    w = w_ref[...]
    h = (jnp.dot(flo, w[:HH, :], preferred_element_type=jnp.float32)
         + jnp.dot(fhi, w[HH:, :], preferred_element_type=jnp.float32))
    h_ref[...] = h
    pa_s = jnp.sum(h * as_ref[...], axis=1, keepdims=True)
    pa_d = jnp.sum(h * ad_ref[...], axis=1, keepdims=True)
    part = jnp.concatenate([pa_s, pa_d], axis=1)

    @pl.when(c == 0)
    def _():
        av_ref[...] = part

    @pl.when(c == 1)
    def _():
        av = av_ref[...] + part
        av_ref[...] = av
        row = i * BLK + lax.broadcasted_iota(jnp.int32, (BLK, 2), 0)
        valid = row < N
        masked = jnp.where(valid, av, NEG_INF)
        cur = jnp.max(masked, axis=0, keepdims=True)
        prev = jnp.where(i == 0, jnp.full((1, 2), NEG_INF), ms_ref[...])
        ms_ref[...] = jnp.maximum(prev, cur)


def _run_l1(x_pad, W1, att_s, att_d):
    return pl.pallas_call(
        _l1_body,
        grid=(NB, 2),
        in_specs=[
            pl.BlockSpec((BLK, F_IN), lambda i, c: (i, 0)),
            pl.BlockSpec((F_IN, HH), lambda i, c: (0, c)),
            pl.BlockSpec((1, HH), lambda i, c: (0, c)),
            pl.BlockSpec((1, HH), lambda i, c: (0, c)),
        ],
        out_specs=[
            pl.BlockSpec((BLK, HH), lambda i, c: (c * NB + i, 0)),
            pl.BlockSpec((BLK, 2), lambda i, c: (i, 0)),
            pl.BlockSpec((1, 2), lambda i, c: (0, 0)),
        ],
        out_shape=[
            jax.ShapeDtypeStruct((2 * NP, HH), jnp.float32),
            jax.ShapeDtypeStruct((NP, 2), jnp.float32),
            jax.ShapeDtypeStruct((1, 2), jnp.float32),
        ],
    )(x_pad, W1, att_s, att_d)


def _run_l2(h_flat, W2, att_s, att_d, scale, shift):
    return pl.pallas_call(
        _l2_body,
        grid=(NB, 2),
        in_specs=[
            pl.BlockSpec((BLK, HH), lambda i, c: (i, 0)),
            pl.BlockSpec((BLK, HH), lambda i, c: (NB + i, 0)),
            pl.BlockSpec((H, HH), lambda i, c: (0, c)),
            pl.BlockSpec((1, HH), lambda i, c: (0, c)),
            pl.BlockSpec((1, HH), lambda i, c: (0, c)),
            pl.BlockSpec((1, H), lambda i, c: (0, 0)),
            pl.BlockSpec((1, H), lambda i, c: (0, 0)),
        ],
        out_specs=[
            pl.BlockSpec((BLK, HH), lambda i, c: (c * NB + i, 0)),
            pl.BlockSpec((BLK, 2), lambda i, c: (i, 0)),
            pl.BlockSpec((1, 2), lambda i, c: (0, 0)),
        ],
        out_shape=[
            jax.ShapeDtypeStruct((2 * NP, HH), jnp.float32),
            jax.ShapeDtypeStruct((NP, 2), jnp.float32),
            jax.ShapeDtypeStruct((1, 2), jnp.float32),
        ],
    )(h_flat, h_flat, W2, att_s, att_d, scale, shift)


def _stats_body(lo_ref, hi_ref, s_ref, ss_ref):
    i = pl.program_id(0)

    @pl.when(i == 0)
    def _():
        s_ref[...] = jnp.zeros_like(s_ref)
        ss_ref[...] = jnp.zeros_like(ss_ref)

    blk = jnp.concatenate([lo_ref[...], hi_ref[...]], axis=1)
    s_ref[...] += jnp.sum(blk, axis=0, keepdims=True)
    ss_ref[...] += jnp.sum(blk * blk, axis=0, keepdims=True)


def _run_stats(flat):
    return pl.pallas_call(
        _stats_body,
        grid=(NB,),
        in_specs=[
            pl.BlockSpec((BLK, HH), lambda i: (i, 0)),
            pl.BlockSpec((BLK, HH), lambda i: (NB + i, 0)),
        ],
        out_specs=[
            pl.BlockSpec((1, H), lambda i: (0, 0)),
            pl.BlockSpec((1, H), lambda i: (0, 0)),
        ],
        out_shape=[
            jax.ShapeDtypeStruct((1, H), jnp.float32),
            jax.ShapeDtypeStruct((1, H), jnp.float32),
        ],
    )(flat, flat)


def _final_body(lo_ref, hi_ref, sc_ref, sh_ref, b_ref, l1w_ref, l1b_ref,
                l2w_ref, l2b_ref, out_ref, plo_ref, phi_ref, cnt_ref):
    i = pl.program_id(0)

    @pl.when(i == 0)
    def _():
        plo_ref[...] = jnp.zeros_like(plo_ref)
        phi_ref[...] = jnp.zeros_like(phi_ref)
        cnt_ref[...] = jnp.zeros_like(cnt_ref)

    flo = jnp.maximum(lo_ref[...] * sc_ref[:, :HH] + sh_ref[:, :HH], 0.0)
    fhi = jnp.maximum(hi_ref[...] * sc_ref[:, HH:] + sh_ref[:, HH:], 0.0)
    bv = b_ref[...].reshape(1, BLK)
    gi = lax.broadcasted_iota(jnp.int32, (G, BLK), 0).astype(jnp.float32)
    oh = (gi == bv).astype(jnp.float32)
    plo_ref[...] += jnp.dot(oh, flo, preferred_element_type=jnp.float32)
    phi_ref[...] += jnp.dot(oh, fhi, preferred_element_type=jnp.float32)
    cnt_ref[...] += jnp.broadcast_to(jnp.sum(oh, axis=1, keepdims=True), (G, HH))

    @pl.when(i == NB - 1)
    def _():
        cc = jnp.maximum(cnt_ref[...], 1.0)
        p_lo = plo_ref[...] / cc
        p_hi = phi_ref[...] / cc
        l1w = l1w_ref[...]
        hm = jnp.maximum(
            jnp.dot(p_lo, l1w[:HH, :], preferred_element_type=jnp.float32)
            + jnp.dot(p_hi, l1w[HH:, :], preferred_element_type=jnp.float32)
            + l1b_ref[...], 0.0)
        out_ref[...] = (jnp.dot(hm, l2w_ref[...],
                                preferred_element_type=jnp.float32)
                        + l2b_ref[...])


def _run_final(flat, scale, shift, batch3, lin1_W, lin1_b, lin2_W, lin2_b):
    return pl.pallas_call(
        _final_body,
        grid=(NB,),
        in_specs=[
            pl.BlockSpec((BLK, HH), lambda i: (i, 0)),
            pl.BlockSpec((BLK, HH), lambda i: (NB + i, 0)),
            pl.BlockSpec((1, H), lambda i: (0, 0)),
            pl.BlockSpec((1, H), lambda i: (0, 0)),
            pl.BlockSpec((1, 1, BLK), lambda i: (i, 0, 0)),
            pl.BlockSpec((H, HH), lambda i: (0, 0)),
            pl.BlockSpec((1, HH), lambda i: (0, 0)),
            pl.BlockSpec((HH, 1), lambda i: (0, 0)),
            pl.BlockSpec((1, 1), lambda i: (0, 0)),
        ],
        out_specs=pl.BlockSpec((G, 1), lambda i: (0, 0)),
        out_shape=jax.ShapeDtypeStruct((G, 1), jnp.float32),
        scratch_shapes=[
            pltpu.VMEM((G, HH), jnp.float32),
            pltpu.VMEM((G, HH), jnp.float32),
            pltpu.VMEM((G, HH), jnp.float32),
        ],
    )(flat, flat, scale, shift, batch3, lin1_W, lin1_b, lin2_W, lin2_b)


SE = 64
SR = 24
SCK = 14
ET = SCK * SR * SE
EPAD = 16 * ET
EROWS = EPAD // SE
FC = 64


def _sc_body(src_hbm, dst_hbm, asrc_hbm, adst_hbm, cvec_hbm, h_hbm, out_hbm,
             as_v, ad_v, src_s, dst_s, w_s, den_c, cvec_v,
             rows0, rows1, out_sh, den_sh, gsem0, gsem1, ssem0, ssem1, dsem):
    c = lax.axis_index("c")
    t = lax.axis_index("s")
    zrows = NP // 16

    pltpu.sync_copy(asrc_hbm, as_v)
    pltpu.sync_copy(adst_hbm, ad_v)
    pltpu.sync_copy(cvec_hbm, cvec_v)
    cv = cvec_v[...]

    z16 = jnp.zeros((16,), jnp.float32)

    def _zero_row(r, carry):
        for q in range(HH // 16):
            rows0[r, pl.ds(q * 16, 16)] = z16
        return carry

    lax.fori_loop(0, FC, _zero_row, 0, unroll=4)
    for k in range(SE // 16):
        den_c[pl.ds(k * 16, 16)] = z16

    def _zchunk(q, carry):
        base = t * zrows + q * FC
        pltpu.sync_copy(rows0.at[pl.ds(0, FC)], out_sh.at[pl.ds(base, FC)])
        pltpu.sync_copy(den_c.at[pl.ds(0, FC)], den_sh.at[pl.ds(base, FC)])
        return carry

    lax.fori_loop(0, zrows // FC, _zchunk, 0)
    plsc.subcore_barrier()

    def _stage_and_ex(s, add_off):
        row0 = t * (SCK * SR) + s * SR
        pltpu.sync_copy(src_hbm.at[pl.ds(row0, SR)], src_s)
        pltpu.sync_copy(dst_hbm.at[pl.ds(row0, SR)], dst_s)
        ebase = t * ET + s * (SR * SE)

        def _row(r, carry):
            for k in range(SE // 16):
                s16 = src_s[r, pl.ds(k * 16, 16)]
                d16 = dst_s[r, pl.ds(k * 16, 16)]
                a1 = plsc.load_gather(as_v, [s16])
                a2 = plsc.load_gather(ad_v, [d16])
                al = a1 + a2
                al = jnp.where(al > 0.0, al, al * jnp.float32(0.2))
                ex = jnp.exp(al - cv)
                eid = (ebase + r * SE + k * 16
                       + lax.broadcasted_iota(jnp.int32, (16,), 0))
                ex = jnp.where(eid < E_REAL, ex, jnp.float32(0.0))
                w_s[r, pl.ds(k * 16, 16)] = ex
                if add_off:
                    src_s[r, pl.ds(k * 16, 16)] = s16 + c * NP
            return carry

        lax.fori_loop(0, SR, _row, 0, unroll=4)

    def _mul_batch(r, rows):
        def _edge(j, carry):
            cb = plsc.load_gather(
                w_s, [jnp.full((16,), r, jnp.int32),
                      jnp.full((16,), j, jnp.int32)])
            for q in range(HH // 16):
                rows[j, pl.ds(q * 16, 16)] = rows[j, pl.ds(q * 16, 16)] * cb
            return carry

        lax.fori_loop(0, SE, _edge, 0, unroll=8)

    def _p2(s, carry):
        _stage_and_ex(s, True)
        pltpu.async_copy(h_hbm.at[src_s.at[0]], rows0, gsem0)
        pltpu.async_copy(h_hbm.at[src_s.at[1]], rows1, gsem1)

        def _fire(r, carry2):
            pltpu.async_copy(w_s.at[r], den_sh.at[dst_s.at[r]], dsem,
                             add=True)
            return carry2

        lax.fori_loop(0, SR, _fire, 0, unroll=8)

        def _pair(rr, carry2):
            r = rr * 2
            pltpu.make_async_copy(h_hbm.at[src_s.at[r]], rows0, gsem0).wait()
            _mul_batch(r, rows0)
            pltpu.async_copy(rows0, out_sh.at[dst_s.at[r]], ssem0, add=True)

            pltpu.make_async_copy(h_hbm.at[src_s.at[r + 1]], rows1,
                                  gsem1).wait()
            _mul_batch(r + 1, rows1)
            pltpu.async_copy(rows1, out_sh.at[dst_s.at[r + 1]], ssem1,
                             add=True)

            pltpu.make_async_copy(rows0, out_sh.at[dst_s.at[r]],
                                  ssem0).wait()

            @pl.when(r + 2 < SR)
            def _():
                pltpu.async_copy(h_hbm.at[src_s.at[r + 2]], rows0, gsem0)

            pltpu.make_async_copy(rows1, out_sh.at[dst_s.at[r + 1]],
                                  ssem1).wait()

            @pl.when(r + 3 < SR)
            def _():
                pltpu.async_copy(h_hbm.at[src_s.at[r + 3]], rows1, gsem1)

            return carry2

        lax.fori_loop(0, SR // 2, _pair, 0)

        def _drain(r, carry2):
            pltpu.make_async_copy(w_s.at[r], den_sh.at[dst_s.at[r]],
                                  dsem).wait()
            return carry2

        lax.fori_loop(0, SR, _drain, 0, unroll=4)
        return carry

    lax.fori_loop(0, SCK, _p2, 0)
    plsc.subcore_barrier()

    def _fin_chunk(q, carry):
        base = t * zrows + q * FC
        pltpu.sync_copy(out_sh.at[pl.ds(base, FC)], rows0.at[pl.ds(0, FC)])
        pltpu.sync_copy(den_sh.at[pl.ds(base, FC)], den_c.at[pl.ds(0, FC)])

        def _rowdiv(r, carry2):
            dv = plsc.load_gather(den_c, [jnp.full((16,), r, jnp.int32)])
            dv = dv + jnp.float32(1e-16)
            for qq in range(HH // 16):
                rows0[r, pl.ds(qq * 16, 16)] = rows0[r, pl.ds(qq * 16, 16)] / dv
            return carry2

        lax.fori_loop(0, FC, _rowdiv, 0, unroll=4)
        pltpu.sync_copy(rows0.at[pl.ds(0, FC)],
                        out_hbm.at[pl.ds(c * NP + base, FC)])
        return carry

    lax.fori_loop(0, zrows // FC, _fin_chunk, 0)


@functools.lru_cache(maxsize=1)
def _sc_agg():
    return functools.partial(
        pl.kernel,
        out_type=jax.ShapeDtypeStruct((2 * NP, HH), jnp.float32),
        mesh=plsc.VectorSubcoreMesh(core_axis_name="c", subcore_axis_name="s"),
        compiler_params=pltpu.CompilerParams(needs_layout_passes=False),
        scratch_types=[
            pltpu.VMEM((NP,), jnp.float32),
            pltpu.VMEM((NP,), jnp.float32),
            pltpu.VMEM((SR, SE), jnp.int32),
            pltpu.VMEM((SR, SE), jnp.int32),
            pltpu.VMEM((SR, SE), jnp.float32),
            pltpu.VMEM((SE,), jnp.float32),
            pltpu.VMEM((16,), jnp.float32),
            pltpu.VMEM((SE, HH), jnp.float32),
            pltpu.VMEM((SE, HH), jnp.float32),
            pltpu.VMEM_SHARED((NP, HH), jnp.float32),
            pltpu.VMEM_SHARED((NP,), jnp.float32),
            pltpu.SemaphoreType.DMA,
            pltpu.SemaphoreType.DMA,
            pltpu.SemaphoreType.DMA,
            pltpu.SemaphoreType.DMA,
            pltpu.SemaphoreType.DMA,
        ],
    )(_sc_body)


def _sc_layer(src2d, dst2d, asrc, adst, cvec, h_flat):
    return _sc_agg()(src2d, dst2d, asrc, adst, cvec, h_flat)


def _bn_params(s, ss, g, b):
    mu = s[0] / N
    var = ss[0] / N - mu * mu
    scale = g / jnp.sqrt(var + 1e-5)
    shift = b - mu * scale
    return scale.reshape(1, H), shift.reshape(1, H)


def kernel(x, edge_index, batch, W1, att_src1, att_dst1, b1, bn1_g, bn1_b,
           W2, att_src2, att_dst2, b2, bn2_g, bn2_b,
           lin1_W, lin1_b, lin2_W, lin2_b):
    loop = jnp.arange(N, dtype=edge_index.dtype)
    src = jnp.concatenate([edge_index[0], loop])
    dst = jnp.concatenate([edge_index[1], loop])
    src2d = jnp.pad(src, (0, EPAD - E_REAL)).reshape(EROWS, SE)
    dst2d = jnp.pad(dst, (0, EPAD - E_REAL)).reshape(EROWS, SE)

    x_pad = jnp.pad(x, ((0, NP - N), (0, 0)))
    batch3 = jnp.pad(batch.astype(jnp.float32), (0, NP - N),
                     constant_values=-1.0).reshape(NB, 1, BLK)

    h1_flat, av1, ms1 = _run_l1(x_pad, W1, att_src1.reshape(1, H),
                                att_dst1.reshape(1, H))
    c1 = jnp.full((16,), ms1[0, 0] + ms1[0, 1], jnp.float32)
    asrc1 = jnp.pad(av1[:N, 0], (0, NP - N))
    adst1 = jnp.pad(av1[:N, 1], (0, NP - N))
    agg1 = _sc_layer(src2d, dst2d, asrc1, adst1, c1, h1_flat)

    s1, ss1 = _run_stats(agg1)
    scale1, shift1 = _bn_params(s1, ss1, bn1_g, bn1_b)

    h2_flat, av2, ms2 = _run_l2(agg1, W2, att_src2.reshape(1, H),
                                att_dst2.reshape(1, H), scale1, shift1)
    c2 = jnp.full((16,), ms2[0, 0] + ms2[0, 1], jnp.float32)
    asrc2 = jnp.pad(av2[:N, 0], (0, NP - N))
    adst2 = jnp.pad(av2[:N, 1], (0, NP - N))
    agg2 = _sc_layer(src2d, dst2d, asrc2, adst2, c2, h2_flat)

    s2, ss2 = _run_stats(agg2)
    scale2, shift2 = _bn_params(s2, ss2, bn2_g, bn2_b)

    out = _run_final(agg2, scale2, shift2, batch3,
                     lin1_W, lin1_b.reshape(1, HH),
                     lin2_W, lin2_b.reshape(1, 1))
    return out.reshape(-1)
